# baseline (speedup 1.0000x reference)
import functools

import jax
import jax.numpy as jnp
from jax import lax
from jax.experimental import pallas as pl
from jax.experimental.pallas import tpu as pltpu
from jax.experimental.pallas import tpu_sc as plsc

N_SPECIES = 512
N_REACTIONS = 65536

_NC = 2
_NS = 16
_LANES = 16
_NW = _NC * _NS
_PER_W = N_REACTIONS // _NW
_TAB = 528

_mesh = plsc.VectorSubcoreMesh(core_axis_name="c", subcore_axis_name="s")


@functools.partial(
    pl.kernel,
    out_type=jax.ShapeDtypeStruct((N_REACTIONS,), jnp.float32),
    mesh=_mesh,
    compiler_params=pltpu.CompilerParams(needs_layout_passes=False),
    scratch_types=[
        pltpu.VMEM((_TAB,), jnp.float32),
        pltpu.VMEM((4, _LANES), jnp.float32),
        pltpu.VMEM((_PER_W,), jnp.float32),
        pltpu.VMEM((_PER_W,), jnp.float32),
        pltpu.VMEM((_PER_W,), jnp.float32),
        pltpu.VMEM((_PER_W,), jnp.float32),
        pltpu.VMEM((_PER_W,), jnp.float32),
        pltpu.VMEM((2 * _PER_W,), jnp.int32),
        pltpu.VMEM((_PER_W,), jnp.float32),
        pltpu.SemaphoreType.DMA,
    ],
)
def _sc_rates(abund_hbm, scal_hbm, alpha_hbm, beta_hbm, gamma_hbm, zeta_hbm,
              fuv_hbm, rm_hbm, rates_hbm,
              tab_v, scal_v, a_v, b_v, g_v, z_v, f_v, rm_v, out_v, sem):
    wid = lax.axis_index("s") * _NC + lax.axis_index("c")
    base = wid * _PER_W
    sl = pl.ds(base, _PER_W)
    copies = [
        pltpu.async_copy(abund_hbm, tab_v.at[pl.ds(0, N_SPECIES)], sem),
        pltpu.async_copy(scal_hbm, scal_v, sem),
        pltpu.async_copy(alpha_hbm.at[sl], a_v, sem),
        pltpu.async_copy(beta_hbm.at[sl], b_v, sem),
        pltpu.async_copy(gamma_hbm.at[sl], g_v, sem),
        pltpu.async_copy(zeta_hbm.at[sl], z_v, sem),
        pltpu.async_copy(fuv_hbm.at[sl], f_v, sem),
        pltpu.async_copy(rm_hbm.at[pl.ds(2 * base, 2 * _PER_W)], rm_v, sem),
    ]
    for c in copies:
        c.wait()
    tab_v[pl.ds(N_SPECIES, _TAB - N_SPECIES)] = jnp.ones(
        (_TAB - N_SPECIES,), jnp.float32)

    log_t = scal_v[0]
    inv_t = scal_v[1]
    cr = scal_v[2]
    fuv = scal_v[3]
    two_iota = 2 * lax.iota(jnp.int32, _LANES)

    @plsc.parallel_loop(0, _PER_W, step=_LANES, unroll=4)
    def body(i):
        s = pl.ds(i, _LANES)
        rate = (a_v[s] * jnp.exp(b_v[s] * log_t - g_v[s] * inv_t)
                + cr * z_v[s] + fuv * f_v[s])
        pair = 2 * i + two_iota
        i0 = plsc.load_gather(rm_v, [pair])
        i1 = plsc.load_gather(rm_v, [pair + 1])
        m0 = plsc.load_gather(tab_v, [i0])
        m1 = plsc.load_gather(tab_v, [i1])
        out_v[s] = rate * m0 * m1

    pltpu.sync_copy(out_v, rates_hbm.at[sl])


_BC = 4096


def _gemv_body(inc_ref, rates_ref, out_ref):
    j = pl.program_id(0)

    @pl.when(j == 0)
    def _init():
        out_ref[...] = jnp.zeros_like(out_ref)

    out_ref[...] += lax.dot_general(
        inc_ref[...], rates_ref[...], (((1,), (1,)), ((), ())),
        preferred_element_type=jnp.float32)


_gemv = pl.pallas_call(
    _gemv_body,
    grid=(N_REACTIONS // _BC,),
    in_specs=[
        pl.BlockSpec((N_SPECIES, _BC), lambda j: (0, j)),
        pl.BlockSpec((1, _BC), lambda j: (0, j)),
    ],
    out_specs=pl.BlockSpec((N_SPECIES, 1), lambda j: (0, 0)),
    out_shape=jax.ShapeDtypeStruct((N_SPECIES, 1), jnp.float32),
)


def kernel(time, abundances, temperature, cr_rate, fuv_rate, incidence,
           reactant_multipliers, alpha, beta, gamma, zeta_coef, fuv_coef):
    del time
    rm_flat = reactant_multipliers.astype(jnp.int32).reshape(2 * N_REACTIONS)
    scal = jnp.stack([
        jnp.full((_LANES,), jnp.log(temperature / 300.0), jnp.float32),
        jnp.full((_LANES,), 1.0 / temperature, jnp.float32),
        jnp.full((_LANES,), cr_rate, jnp.float32),
        jnp.full((_LANES,), fuv_rate, jnp.float32),
    ])
    rates = _sc_rates(abundances, scal, alpha, beta, gamma, zeta_coef,
                      fuv_coef, rm_flat)
    out = _gemv(incidence, rates.reshape(1, N_REACTIONS))
    return out.reshape(N_SPECIES)


# speedup vs baseline: 1.0015x; 1.0015x over previous
import functools

import jax
import jax.numpy as jnp
from jax import lax
from jax.experimental import pallas as pl
from jax.experimental.pallas import tpu as pltpu
from jax.experimental.pallas import tpu_sc as plsc

N_SPECIES = 512
N_REACTIONS = 65536

_NC = 2
_NS = 16
_LANES = 16
_NW = _NC * _NS
_PER_W = N_REACTIONS // _NW
_TAB = 528

_mesh = plsc.VectorSubcoreMesh(core_axis_name="c", subcore_axis_name="s")


@functools.partial(
    pl.kernel,
    out_type=jax.ShapeDtypeStruct((N_REACTIONS,), jnp.float32),
    mesh=_mesh,
    compiler_params=pltpu.CompilerParams(needs_layout_passes=False),
    scratch_types=[
        pltpu.VMEM((_TAB,), jnp.float32),
        pltpu.VMEM((4, _LANES), jnp.float32),
        pltpu.VMEM((_PER_W,), jnp.float32),
        pltpu.VMEM((_PER_W,), jnp.float32),
        pltpu.VMEM((_PER_W,), jnp.float32),
        pltpu.VMEM((_PER_W,), jnp.float32),
        pltpu.VMEM((_PER_W,), jnp.float32),
        pltpu.VMEM((2 * _PER_W,), jnp.int32),
        pltpu.VMEM((_PER_W,), jnp.float32),
        pltpu.SemaphoreType.DMA,
    ],
)
def _sc_rates(abund_hbm, scal_hbm, alpha_hbm, beta_hbm, gamma_hbm, zeta_hbm,
              fuv_hbm, rm_hbm, rates_hbm,
              tab_v, scal_v, a_v, b_v, g_v, z_v, f_v, rm_v, out_v, sem):
    wid = lax.axis_index("s") * _NC + lax.axis_index("c")
    base = wid * _PER_W
    sl = pl.ds(base, _PER_W)
    copies = [
        pltpu.async_copy(abund_hbm, tab_v.at[pl.ds(0, N_SPECIES)], sem),
        pltpu.async_copy(scal_hbm, scal_v, sem),
        pltpu.async_copy(alpha_hbm.at[sl], a_v, sem),
        pltpu.async_copy(beta_hbm.at[sl], b_v, sem),
        pltpu.async_copy(gamma_hbm.at[sl], g_v, sem),
        pltpu.async_copy(zeta_hbm.at[sl], z_v, sem),
        pltpu.async_copy(fuv_hbm.at[sl], f_v, sem),
        pltpu.async_copy(rm_hbm.at[pl.ds(2 * base, 2 * _PER_W)], rm_v, sem),
    ]
    for c in copies:
        c.wait()
    tab_v[pl.ds(N_SPECIES, _TAB - N_SPECIES)] = jnp.ones(
        (_TAB - N_SPECIES,), jnp.float32)

    log_t = scal_v[0]
    inv_t = scal_v[1]
    cr = scal_v[2]
    fuv = scal_v[3]
    two_iota = 2 * lax.iota(jnp.int32, _LANES)

    @plsc.parallel_loop(0, _PER_W, step=_LANES, unroll=2)
    def body(i):
        s = pl.ds(i, _LANES)
        rate = (a_v[s] * jnp.exp(b_v[s] * log_t - g_v[s] * inv_t)
                + cr * z_v[s] + fuv * f_v[s])
        pair = 2 * i + two_iota
        i0 = plsc.load_gather(rm_v, [pair])
        i1 = plsc.load_gather(rm_v, [pair + 1])
        m0 = plsc.load_gather(tab_v, [i0])
        m1 = plsc.load_gather(tab_v, [i1])
        out_v[s] = rate * m0 * m1

    pltpu.sync_copy(out_v, rates_hbm.at[sl])


_BC = 4096


def _gemv_body(inc_ref, rates_ref, out_ref):
    j = pl.program_id(0)

    @pl.when(j == 0)
    def _init():
        out_ref[...] = jnp.zeros_like(out_ref)

    out_ref[...] += lax.dot_general(
        inc_ref[...], rates_ref[...], (((1,), (1,)), ((), ())),
        preferred_element_type=jnp.float32)


_gemv = pl.pallas_call(
    _gemv_body,
    grid=(N_REACTIONS // _BC,),
    in_specs=[
        pl.BlockSpec((N_SPECIES, _BC), lambda j: (0, j)),
        pl.BlockSpec((1, _BC), lambda j: (0, j)),
    ],
    out_specs=pl.BlockSpec((N_SPECIES, 1), lambda j: (0, 0)),
    out_shape=jax.ShapeDtypeStruct((N_SPECIES, 1), jnp.float32),
)


def kernel(time, abundances, temperature, cr_rate, fuv_rate, incidence,
           reactant_multipliers, alpha, beta, gamma, zeta_coef, fuv_coef):
    del time
    rm_flat = reactant_multipliers.astype(jnp.int32).reshape(2 * N_REACTIONS)
    scal = jnp.stack([
        jnp.full((_LANES,), jnp.log(temperature / 300.0), jnp.float32),
        jnp.full((_LANES,), 1.0 / temperature, jnp.float32),
        jnp.full((_LANES,), cr_rate, jnp.float32),
        jnp.full((_LANES,), fuv_rate, jnp.float32),
    ])
    rates = _sc_rates(abundances, scal, alpha, beta, gamma, zeta_coef,
                      fuv_coef, rm_flat)
    out = _gemv(incidence, rates.reshape(1, N_REACTIONS))
    return out.reshape(N_SPECIES)


# speedup vs baseline: 1.5598x; 1.5574x over previous
import functools

import jax
import jax.numpy as jnp
from jax import lax
from jax.experimental import pallas as pl
from jax.experimental.pallas import tpu as pltpu
from jax.experimental.pallas import tpu_sc as plsc

N_SPECIES = 512
N_REACTIONS = 65536

_NC = 2
_NS = 16
_LANES = 16
_NW = _NC * _NS
_PER_W = N_REACTIONS // _NW
_TAB = 528

_mesh = plsc.VectorSubcoreMesh(core_axis_name="c", subcore_axis_name="s")


@functools.partial(
    pl.kernel,
    out_type=jax.ShapeDtypeStruct((N_REACTIONS,), jnp.float32),
    mesh=_mesh,
    compiler_params=pltpu.CompilerParams(needs_layout_passes=False),
    scratch_types=[
        pltpu.VMEM((_TAB,), jnp.float32),
        pltpu.VMEM((4, _LANES), jnp.float32),
        pltpu.VMEM((_PER_W,), jnp.float32),
        pltpu.VMEM((_PER_W,), jnp.float32),
        pltpu.VMEM((_PER_W,), jnp.float32),
        pltpu.VMEM((_PER_W,), jnp.float32),
        pltpu.VMEM((_PER_W,), jnp.float32),
        pltpu.VMEM((_PER_W,), jnp.int32),
        pltpu.VMEM((_PER_W,), jnp.int32),
        pltpu.VMEM((_PER_W,), jnp.float32),
        pltpu.SemaphoreType.DMA,
    ],
)
def _sc_rates(abund_hbm, scal_hbm, alpha_hbm, beta_hbm, gamma_hbm, zeta_hbm,
              fuv_hbm, rm0_hbm, rm1_hbm, rates_hbm,
              tab_v, scal_v, a_v, b_v, g_v, z_v, f_v, i0_v, i1_v, out_v, sem):
    wid = lax.axis_index("s") * _NC + lax.axis_index("c")
    base = wid * _PER_W
    sl = pl.ds(base, _PER_W)
    copies = [
        pltpu.async_copy(abund_hbm, tab_v.at[pl.ds(0, N_SPECIES)], sem),
        pltpu.async_copy(scal_hbm, scal_v, sem),
        pltpu.async_copy(alpha_hbm.at[sl], a_v, sem),
        pltpu.async_copy(beta_hbm.at[sl], b_v, sem),
        pltpu.async_copy(gamma_hbm.at[sl], g_v, sem),
        pltpu.async_copy(zeta_hbm.at[sl], z_v, sem),
        pltpu.async_copy(fuv_hbm.at[sl], f_v, sem),
        pltpu.async_copy(rm0_hbm.at[sl], i0_v, sem),
        pltpu.async_copy(rm1_hbm.at[sl], i1_v, sem),
    ]
    for c in copies:
        c.wait()
    tab_v[pl.ds(N_SPECIES, _TAB - N_SPECIES)] = jnp.ones(
        (_TAB - N_SPECIES,), jnp.float32)

    log_t = scal_v[0]
    inv_t = scal_v[1]
    cr = scal_v[2]
    fuv = scal_v[3]

    @plsc.parallel_loop(0, _PER_W, step=_LANES, unroll=4)
    def body(i):
        s = pl.ds(i, _LANES)
        rate = (a_v[s] * jnp.exp(b_v[s] * log_t - g_v[s] * inv_t)
                + cr * z_v[s] + fuv * f_v[s])
        m0 = plsc.load_gather(tab_v, [i0_v[s]])
        m1 = plsc.load_gather(tab_v, [i1_v[s]])
        out_v[s] = rate * m0 * m1

    pltpu.sync_copy(out_v, rates_hbm.at[sl])


_BC = 4096


def _gemv_body(inc_ref, rates_ref, out_ref):
    j = pl.program_id(0)

    @pl.when(j == 0)
    def _init():
        out_ref[...] = jnp.zeros_like(out_ref)

    out_ref[...] += lax.dot_general(
        inc_ref[...], rates_ref[...], (((1,), (1,)), ((), ())),
        preferred_element_type=jnp.float32)


_gemv = pl.pallas_call(
    _gemv_body,
    grid=(N_REACTIONS // _BC,),
    in_specs=[
        pl.BlockSpec((N_SPECIES, _BC), lambda j: (0, j)),
        pl.BlockSpec((1, _BC), lambda j: (0, j)),
    ],
    out_specs=pl.BlockSpec((N_SPECIES, 1), lambda j: (0, 0)),
    out_shape=jax.ShapeDtypeStruct((N_SPECIES, 1), jnp.float32),
)


def kernel(time, abundances, temperature, cr_rate, fuv_rate, incidence,
           reactant_multipliers, alpha, beta, gamma, zeta_coef, fuv_coef):
    del time
    rm = reactant_multipliers.astype(jnp.int32)
    rm0 = rm[:, 0]
    rm1 = rm[:, 1]
    scal = jnp.stack([
        jnp.full((_LANES,), jnp.log(temperature / 300.0), jnp.float32),
        jnp.full((_LANES,), 1.0 / temperature, jnp.float32),
        jnp.full((_LANES,), cr_rate, jnp.float32),
        jnp.full((_LANES,), fuv_rate, jnp.float32),
    ])
    rates = _sc_rates(abundances, scal, alpha, beta, gamma, zeta_coef,
                      fuv_coef, rm0, rm1)
    out = _gemv(incidence, rates.reshape(1, N_REACTIONS))
    return out.reshape(N_SPECIES)
